# baseline (speedup 1.0000x reference)
import functools

import jax
import jax.numpy as jnp
from jax import lax
from jax.experimental import pallas as pl
from jax.experimental.pallas import tpu as pltpu
from jax.experimental.pallas import tpu_sc as plsc

BATCH = 16384
DIM = 64
NUM_WORKERS = 32
B_PER_W = BATCH // NUM_WORKERS
CHUNK = 128
NCHUNK = B_PER_W // CHUNK


def kernel(x, weight):
    mesh = plsc.VectorSubcoreMesh(core_axis_name="c", subcore_axis_name="s")

    @functools.partial(
        pl.kernel,
        mesh=mesh,
        out_type=jax.ShapeDtypeStruct((BATCH, DIM), jnp.float32),
        scratch_types=[
            pltpu.VMEM((B_PER_W,), jnp.int32),
            pltpu.VMEM((B_PER_W, DIM), jnp.float32),
            pltpu.SemaphoreType.DMA,
        ],
        compiler_params=pltpu.CompilerParams(use_tc_tiling_on_sc=False),
    )
    def body(x_hbm, w_hbm, out_hbm, idx_v, rows_v, sem):
        wid = lax.axis_index("s") * 2 + lax.axis_index("c")
        base = wid * B_PER_W
        pltpu.sync_copy(x_hbm.at[pl.ds(base, B_PER_W)], idx_v)
        copies = []
        for c in range(NCHUNK):
            copies.append(
                pltpu.async_copy(
                    w_hbm.at[idx_v.at[pl.ds(c * CHUNK, CHUNK)]],
                    rows_v.at[pl.ds(c * CHUNK, CHUNK)],
                    sem,
                )
            )
        for cp in copies:
            cp.wait()
        pltpu.sync_copy(rows_v, out_hbm.at[pl.ds(base, B_PER_W)])

    return body(x.astype(jnp.int32), weight)


# speedup vs baseline: 1.8260x; 1.8260x over previous
import functools

import jax
import jax.numpy as jnp
from jax import lax
from jax.experimental import pallas as pl
from jax.experimental.pallas import tpu as pltpu
from jax.experimental.pallas import tpu_sc as plsc

NUM_EMB = 1000000
DIM = 64
BATCH = 16384
NUM_WORKERS = 32
B_PER_W = BATCH // NUM_WORKERS
NGRP = B_PER_W // 16
MAX_T = NUM_EMB // 128 - 1
TAIL_LO = NUM_EMB - DIM
TAIL_START = NUM_EMB - 128


def kernel(x, weight):
    wt = weight.T
    tail = lax.slice(wt, (0, TAIL_START), (DIM, NUM_EMB))
    mesh = plsc.VectorSubcoreMesh(core_axis_name="c", subcore_axis_name="s")

    @functools.partial(
        pl.kernel,
        mesh=mesh,
        out_type=jax.ShapeDtypeStruct((DIM, BATCH), jnp.float32),
        scratch_types=[
            pltpu.VMEM((B_PER_W,), jnp.int32),
            pltpu.VMEM((DIM, 128), jnp.float32),
            pltpu.VMEM((DIM, 128), jnp.float32),
            pltpu.VMEM((DIM, 128), jnp.float32),
            pltpu.VMEM((DIM, B_PER_W), jnp.float32),
            pltpu.SemaphoreType.DMA,
            pltpu.SemaphoreType.DMA,
        ],
        compiler_params=pltpu.CompilerParams(needs_layout_passes=False),
    )
    def body(x_hbm, w_hbm, tail_hbm, out_hbm, xi_v, slab0, slab1,
             tail_v, stage, sem0, sem1):
        wid = lax.axis_index("s") * 2 + lax.axis_index("c")
        base = wid * B_PER_W
        pltpu.sync_copy(x_hbm.at[pl.ds(base, B_PER_W)], xi_v)
        pltpu.sync_copy(tail_hbm, tail_v)

        slabs = (slab0, slab1)
        sems = (sem0, sem1)
        dvecs = [lax.iota(jnp.int32, 16) + 16 * q for q in range(4)]

        def window_start(i):
            t = jnp.minimum(lax.shift_right_logical(i, 7), jnp.int32(MAX_T))
            return pl.multiple_of(t * 128, 128)

        def fire(i, b):
            return pltpu.async_copy(
                w_hbm.at[:, pl.ds(window_start(i), 128)], slabs[b], sems[b]
            )

        first = xi_v[pl.ds(0, 16)]
        fire(first[0], 0)
        fire(first[1], 1)

        @pl.loop(0, NGRP)
        def grp_loop(g):
            cur = xi_v[pl.ds(g * 16, 16)]
            nxt = xi_v[pl.ds(jnp.minimum(g + 1, NGRP - 1) * 16, 16)]
            for lane in range(16):
                k = g * 16 + lane
                b = lane % 2
                pltpu.make_async_copy(
                    w_hbm.at[:, pl.ds(0, 128)], slabs[b], sems[b]
                ).wait()
                i = cur[lane]
                col = jnp.minimum(i - window_start(i), jnp.int32(127))
                csplat = jnp.full((16,), col, jnp.int32)
                ksplat = jnp.full((16,), k, jnp.int32)
                for q in range(4):
                    v = plsc.load_gather(slabs[b], [dvecs[q], csplat])
                    plsc.store_scatter(stage, [dvecs[q], ksplat], v)
                nxt_i = cur[lane + 2] if lane < 14 else nxt[lane - 14]
                fire(nxt_i, b)

        for b in range(2):
            pltpu.make_async_copy(
                w_hbm.at[:, pl.ds(0, 128)], slabs[b], sems[b]
            ).wait()

        @pl.loop(0, NGRP)
        def tail_loop(g):
            cur = xi_v[pl.ds(g * 16, 16)]
            for lane in range(16):
                k = g * 16 + lane
                i = cur[lane]

                @pl.when(i >= TAIL_LO)
                def _():
                    col = i - TAIL_START
                    csplat = jnp.full((16,), col, jnp.int32)
                    ksplat = jnp.full((16,), k, jnp.int32)
                    for q in range(4):
                        v = plsc.load_gather(tail_v, [dvecs[q], csplat])
                        plsc.store_scatter(stage, [dvecs[q], ksplat], v)

        pltpu.sync_copy(stage, out_hbm.at[:, pl.ds(base, B_PER_W)])

    return body(x.astype(jnp.int32), wt, tail).T


# speedup vs baseline: 2.8401x; 1.5554x over previous
import functools

import jax
import jax.numpy as jnp
from jax import lax
from jax.experimental import pallas as pl
from jax.experimental.pallas import tpu as pltpu
from jax.experimental.pallas import tpu_sc as plsc

NUM_EMB = 1000000
DIM = 64
BATCH = 16384
NUM_WORKERS = 32
B_PER_W = BATCH // NUM_WORKERS
NGRP = B_PER_W // 16
MAX_T = NUM_EMB // 128 - 1
TAIL_LO = NUM_EMB - DIM
TAIL_START = NUM_EMB - 128


def kernel(x, weight):
    wt = weight.T
    tail = lax.slice(wt, (0, TAIL_START), (DIM, NUM_EMB))
    mesh = plsc.VectorSubcoreMesh(core_axis_name="c", subcore_axis_name="s")

    @functools.partial(
        pl.kernel,
        mesh=mesh,
        out_type=jax.ShapeDtypeStruct((DIM, BATCH), jnp.float32),
        scratch_types=[
            pltpu.VMEM((B_PER_W,), jnp.int32),
            *[pltpu.VMEM((DIM, 128), jnp.float32) for _ in range(8)],
            pltpu.VMEM((DIM, 128), jnp.float32),
            pltpu.VMEM((DIM, B_PER_W), jnp.float32),
            *[pltpu.SemaphoreType.DMA for _ in range(8)],
        ],
        compiler_params=pltpu.CompilerParams(needs_layout_passes=False),
    )
    def body(x_hbm, w_hbm, tail_hbm, out_hbm, xi_v, *rest):
        slabs = rest[0:8]
        tail_v = rest[8]
        stage = rest[9]
        sems = rest[10:18]
        wid = lax.axis_index("s") * 2 + lax.axis_index("c")
        base = wid * B_PER_W
        pltpu.sync_copy(x_hbm.at[pl.ds(base, B_PER_W)], xi_v)
        pltpu.sync_copy(tail_hbm, tail_v)

        dvecs = [lax.iota(jnp.int32, 16) + 16 * q for q in range(4)]

        def window_start(i):
            t = jnp.minimum(lax.shift_right_logical(i, 7), jnp.int32(MAX_T))
            return pl.multiple_of(t * 128, 128)

        def fire(i, b):
            return pltpu.async_copy(
                w_hbm.at[:, pl.ds(window_start(i), 128)], slabs[b], sems[b]
            )

        first = xi_v[pl.ds(0, 16)]
        for b in range(8):
            fire(first[b], b)

        @pl.loop(0, NGRP)
        def grp_loop(g):
            cur = xi_v[pl.ds(g * 16, 16)]
            nxt = xi_v[pl.ds(jnp.minimum(g + 1, NGRP - 1) * 16, 16)]
            for lane in range(16):
                k = g * 16 + lane
                b = lane % 8
                pltpu.make_async_copy(
                    w_hbm.at[:, pl.ds(0, 128)], slabs[b], sems[b]
                ).wait()
                i = cur[lane]
                col = jnp.minimum(i - window_start(i), jnp.int32(127))
                csplat = jnp.full((16,), col, jnp.int32)
                ksplat = jnp.full((16,), k, jnp.int32)
                for q in range(4):
                    v = plsc.load_gather(slabs[b], [dvecs[q], csplat])
                    plsc.store_scatter(stage, [dvecs[q], ksplat], v)
                nxt_i = cur[lane + 8] if lane < 8 else nxt[lane - 8]
                fire(nxt_i, b)

        for b in range(8):
            pltpu.make_async_copy(
                w_hbm.at[:, pl.ds(0, 128)], slabs[b], sems[b]
            ).wait()

        @pl.loop(0, NGRP)
        def tail_loop(g):
            cur = xi_v[pl.ds(g * 16, 16)]
            for lane in range(16):
                k = g * 16 + lane
                i = cur[lane]

                @pl.when(i >= TAIL_LO)
                def _():
                    col = i - TAIL_START
                    csplat = jnp.full((16,), col, jnp.int32)
                    ksplat = jnp.full((16,), k, jnp.int32)
                    for q in range(4):
                        v = plsc.load_gather(tail_v, [dvecs[q], csplat])
                        plsc.store_scatter(stage, [dvecs[q], ksplat], v)

        pltpu.sync_copy(stage, out_hbm.at[:, pl.ds(base, B_PER_W)])

    return body(x.astype(jnp.int32), wt, tail).T
